# baseline (speedup 1.0000x reference)
import functools

import jax
import jax.numpy as jnp
from jax import lax
from jax.experimental import pallas as pl
from jax.experimental.pallas import tpu as pltpu
from jax.experimental.pallas import tpu_sc as plsc

B = 50000
D = 128
DW = D // 2
S = 10
NW = 32
BLK = 64
NBLK = 25
CHUNK = BLK * NBLK
B_PAD = NW * CHUNK
NIDX = NBLK * (S + 1)


@functools.cache
def _make_sc_gather():
    @functools.partial(
        pl.kernel,
        out_type=[
            jax.ShapeDtypeStruct((B_PAD, DW), jnp.int32),
            jax.ShapeDtypeStruct((B_PAD, DW), jnp.int32),
        ],
        mesh=plsc.VectorSubcoreMesh(core_axis_name="c", subcore_axis_name="s"),
        compiler_params=pltpu.CompilerParams(
            use_tc_tiling_on_sc=False, needs_layout_passes=False),
        scratch_types=[
            pltpu.VMEM((NIDX, BLK), jnp.int32),
            pltpu.VMEM((2, BLK, DW), jnp.int32),
            pltpu.VMEM((2, S, BLK, DW), jnp.int32),
            pltpu.SemaphoreType.DMA,
            pltpu.SemaphoreType.DMA,
            pltpu.SemaphoreType.DMA,
            pltpu.SemaphoreType.DMA,
            pltpu.SemaphoreType.DMA,
            pltpu.SemaphoreType.DMA,
            pltpu.SemaphoreType.DMA,
            pltpu.SemaphoreType.DMA,
        ],
    )
    def _sc_gather(idx_hbm, feat_hbm, self_out, sum_out,
                   idx_v, selfb, nb, sgs0, sgs1, sos0, sos1,
                   sgn0, sgn1, son0, son1):
        sg_self = (sgs0, sgs1)
        so_self = (sos0, sos1)
        sg_n = (sgn0, sgn1)
        so_n = (son0, son1)
        wid = lax.axis_index("s") * 2 + lax.axis_index("c")
        wbase = wid * CHUNK

        pltpu.sync_copy(idx_hbm.at[wid], idx_v)

        def fire_block(b):
            p = b & 1
            ds = pltpu.async_copy(
                feat_hbm.at[idx_v.at[b * (S + 1)]], selfb.at[p], sg_self[p])
            da = [
                pltpu.async_copy(
                    feat_hbm.at[idx_v.at[b * (S + 1) + 1 + j]], nb.at[p, j],
                    sg_n[p])
                for j in range(S)
            ]
            return ds, da

        def reduce_block(p):
            def row(r, c2):
                for c in range(DW // 16):
                    sl = pl.ds(c * 16, 16)
                    a = plsc.bitcast(nb[p, 0, r, sl], jnp.bfloat16)
                    for j in range(1, S):
                        a = a + plsc.bitcast(nb[p, j, r, sl], jnp.bfloat16)
                    nb[p, 0, r, sl] = plsc.bitcast(a, jnp.int32)
                return c2

            lax.fori_loop(0, BLK, row, 0)

        def retire_block(b, gathers):
            p = b & 1
            base = wbase + b * BLK
            ds, da = gathers
            ds.wait()
            os = pltpu.async_copy(selfb.at[p], self_out.at[pl.ds(base, BLK)],
                                  so_self[p])
            for d in da:
                d.wait()
            reduce_block(p)
            on = pltpu.async_copy(nb.at[p, 0], sum_out.at[pl.ds(base, BLK)],
                                  so_n[p])
            return os, on

        outs_self = [None, None]
        outs_n = [None, None]
        gathers = fire_block(0)
        for b in range(NBLK):
            nxt = None
            if b + 1 < NBLK:
                p = (b + 1) & 1
                if outs_self[p] is not None:
                    outs_self[p].wait()
                if outs_n[p] is not None:
                    outs_n[p].wait()
                nxt = fire_block(b + 1)
            os, on = retire_block(b, gathers)
            outs_self[b & 1] = os
            outs_n[b & 1] = on
            gathers = nxt
        for dd in outs_self:
            dd.wait()
        for dd in outs_n:
            dd.wait()

    return _sc_gather


MB = 512


def _mm_body(self_ref, sum_ref, w_ref, o_ref):
    w1 = w_ref[:D, :]
    w2 = w_ref[D:, :]
    acc = jnp.dot(self_ref[...], w1, preferred_element_type=jnp.float32)
    acc += jnp.dot(sum_ref[...], w2, preferred_element_type=jnp.float32)
    o_ref[...] = jnp.maximum(acc, 0.0)


def _as_bf16(x_i32):
    return lax.bitcast_convert_type(x_i32, jnp.bfloat16).reshape(-1, D)


def kernel(nodes, neigh_idx, features, weight):
    pad = B_PAD - B
    nodes_r = jnp.pad(nodes, (0, pad)).reshape(NW, NBLK, 1, BLK)
    neigh_r = (jnp.pad(neigh_idx, ((0, pad), (0, 0)))
               .reshape(NW, NBLK, BLK, S)
               .transpose(0, 1, 3, 2))
    idx_all = jnp.concatenate([nodes_r, neigh_r], axis=2).reshape(NW, NIDX, BLK)
    feat_bf = features.astype(jnp.bfloat16)
    feat_i32 = lax.bitcast_convert_type(
        feat_bf.reshape(features.shape[0], DW, 2), jnp.int32)
    w_bf = jnp.concatenate(
        [weight[:D], weight[D:] * jnp.float32(1.0 / S)], axis=0
    ).astype(jnp.bfloat16)
    self_i32, sum_i32 = _make_sc_gather()(idx_all, feat_i32)
    out = pl.pallas_call(
        _mm_body,
        grid=(50176 // MB,),
        in_specs=[
            pl.BlockSpec((MB, D), lambda i: (i, 0)),
            pl.BlockSpec((MB, D), lambda i: (i, 0)),
            pl.BlockSpec((2 * D, D), lambda i: (0, 0)),
        ],
        out_specs=pl.BlockSpec((MB, D), lambda i: (i, 0)),
        out_shape=jax.ShapeDtypeStruct((B, D), jnp.float32),
    )(_as_bf16(self_i32), _as_bf16(sum_i32), w_bf)
    return out


# speedup vs baseline: 4.1237x; 4.1237x over previous
import functools

import jax
import jax.numpy as jnp
from jax import lax
from jax.experimental import pallas as pl
from jax.experimental.pallas import tpu as pltpu
from jax.experimental.pallas import tpu_sc as plsc

B = 50000
D = 128
S = 10
NW = 32
BLK = 112
MB = 512


@functools.cache
def _make_sc_gather(nblk):
    chunk = BLK * nblk
    b_pad = NW * chunk
    nidx = nblk * (S + 1)

    @functools.partial(
        pl.kernel,
        out_type=[
            jax.ShapeDtypeStruct((b_pad, D), jnp.float32),
            jax.ShapeDtypeStruct((b_pad, D), jnp.float32),
        ],
        mesh=plsc.VectorSubcoreMesh(core_axis_name="c", subcore_axis_name="s"),
        scratch_types=[
            pltpu.VMEM((nidx, BLK), jnp.int32),
            pltpu.VMEM((2, BLK, D), jnp.float32),
            pltpu.VMEM((2, BLK, D), jnp.float32),
            pltpu.SemaphoreType.DMA,
            pltpu.SemaphoreType.DMA,
            pltpu.SemaphoreType.DMA,
            pltpu.SemaphoreType.DMA,
            pltpu.SemaphoreType.DMA,
            pltpu.SemaphoreType.DMA,
            pltpu.SemaphoreType.DMA,
            pltpu.SemaphoreType.DMA,
        ],
    )
    def _sc_gather(idx_hbm, feat_hbm, self_out, sum_out,
                   idx_v, selfb, accb, sgs0, sgs1, sga0, sga1,
                   sos0, sos1, soa0, soa1):
        sg_self = (sgs0, sgs1)
        sg_acc = (sga0, sga1)
        so_self = (sos0, sos1)
        so_acc = (soa0, soa1)
        wid = lax.axis_index("s") * 2 + lax.axis_index("c")
        wbase = wid * chunk

        pltpu.sync_copy(idx_hbm.at[wid], idx_v)

        def fire_block(b):
            p = b & 1
            ds = pltpu.async_copy(
                feat_hbm.at[idx_v.at[b * (S + 1)]], selfb.at[p], sg_self[p])
            da = [pltpu.async_copy(
                feat_hbm.at[idx_v.at[b * (S + 1) + 1]], accb.at[p], sg_acc[p])]
            da += [
                pltpu.async_copy(
                    feat_hbm.at[idx_v.at[b * (S + 1) + 1 + j]], accb.at[p],
                    sg_acc[p], add=True)
                for j in range(1, S)
            ]
            return ds, da

        def retire_block(b, gathers):
            p = b & 1
            base = wbase + b * BLK
            ds, da = gathers
            ds.wait()
            os = pltpu.async_copy(selfb.at[p], self_out.at[pl.ds(base, BLK)],
                                  so_self[p])
            for d in da:
                d.wait()
            oa = pltpu.async_copy(accb.at[p], sum_out.at[pl.ds(base, BLK)],
                                  so_acc[p])
            return os, oa

        outs = [None, None]
        gathers = fire_block(0)
        for b in range(nblk):
            nxt = None
            if b + 1 < nblk:
                p = (b + 1) & 1
                if outs[p] is not None:
                    outs[p][0].wait()
                    outs[p][1].wait()
                nxt = fire_block(b + 1)
            outs[b & 1] = retire_block(b, gathers)
            gathers = nxt
        outs[0][0].wait()
        outs[0][1].wait()
        outs[1][0].wait()
        outs[1][1].wait()

    return _sc_gather


def _mm_body(self_ref, sum_ref, w_ref, o_ref):
    w1 = w_ref[:D, :]
    w2 = w_ref[D:, :]
    x2 = sum_ref[...] * jnp.float32(1.0 / S)
    acc = jnp.dot(self_ref[...], w1, preferred_element_type=jnp.float32)
    acc += jnp.dot(x2, w2, preferred_element_type=jnp.float32)
    o_ref[...] = jnp.maximum(acc, 0.0)


def _interleave_idx(nodes_h, neigh_h, nblk):
    b_pad = NW * BLK * nblk
    pad = b_pad - nodes_h.shape[0]
    nodes_r = jnp.pad(nodes_h, (0, pad)).reshape(NW, nblk, 1, BLK)
    neigh_r = (jnp.pad(neigh_h, ((0, pad), (0, 0)))
               .reshape(NW, nblk, BLK, S)
               .transpose(0, 1, 3, 2))
    return jnp.concatenate([nodes_r, neigh_r], axis=2).reshape(
        NW, nblk * (S + 1), BLK)


def _half(nodes_h, neigh_h, features, weight, nblk, nout):
    idx_all = _interleave_idx(nodes_h, neigh_h, nblk)
    self_feats, neigh_sum = _make_sc_gather(nblk)(idx_all, features)
    b_pad = NW * BLK * nblk
    grid = (nout + MB - 1) // MB
    return pl.pallas_call(
        _mm_body,
        grid=(grid,),
        in_specs=[
            pl.BlockSpec((MB, D), lambda i: (i, 0)),
            pl.BlockSpec((MB, D), lambda i: (i, 0)),
            pl.BlockSpec((2 * D, D), lambda i: (0, 0)),
        ],
        out_specs=pl.BlockSpec((MB, D), lambda i: (i, 0)),
        out_shape=jax.ShapeDtypeStruct((nout, D), jnp.float32),
    )(self_feats, neigh_sum, weight)


def kernel(nodes, neigh_idx, features, weight):
    hb = B // 2
    nblk = 7
    o0 = _half(nodes[:hb], neigh_idx[:hb], features, weight, nblk, hb)
    o1 = _half(nodes[hb:], neigh_idx[hb:], features, weight, nblk, hb)
    return jnp.concatenate([o0, o1], axis=0)
